# Initial kernel scaffold; baseline (speedup 1.0000x reference)
import jax
import jax.numpy as jnp
from jax.experimental import pallas as pl


def kernel(idx, targets, token_embedding_table):
    raise NotImplementedError("write your pallas kernel here")


# speedup vs baseline: 1.0146x; 1.0146x over previous
import functools

import jax
import jax.numpy as jnp
from jax import lax
from jax.experimental import pallas as pl
from jax.experimental.pallas import tpu as pltpu
from jax.experimental.pallas import tpu_sc as plsc

NUM_WORKERS = 32
CHUNK = 64


def kernel(idx, targets, token_embedding_table):
    del targets
    B, L = idx.shape
    V, D = token_embedding_table.shape
    N = B * L
    per_w = N // NUM_WORKERS
    n_ch = per_w // CHUNK
    assert per_w * NUM_WORKERS == N and n_ch * CHUNK == per_w

    flat_idx = idx.astype(jnp.int32).reshape(NUM_WORKERS, n_ch, CHUNK)

    mesh = plsc.VectorSubcoreMesh(core_axis_name="c", subcore_axis_name="s")

    @functools.partial(
        pl.kernel,
        out_type=jax.ShapeDtypeStruct((N, D), jnp.float32),
        mesh=mesh,
        compiler_params=pltpu.CompilerParams(use_tc_tiling_on_sc=False),
        scratch_types=[
            pltpu.VMEM((n_ch, CHUNK), jnp.int32),
            pltpu.VMEM((CHUNK, D), jnp.float32),
            pltpu.SemaphoreType.DMA,
            pltpu.SemaphoreType.DMA,
        ],
    )
    def gather_kernel(table_hbm, idx_hbm, out_hbm, idx_v, rows_v, g_sem, w_sem):
        wid = lax.axis_index("s") * 2 + lax.axis_index("c")
        base = wid * per_w
        pltpu.sync_copy(idx_hbm.at[wid], idx_v)

        @pl.loop(0, n_ch)
        def _chunk(j):
            pltpu.async_copy(table_hbm.at[idx_v.at[j]], rows_v, g_sem).wait()
            pltpu.async_copy(
                rows_v, out_hbm.at[pl.ds(base + j * CHUNK, CHUNK)], w_sem
            ).wait()

    out = gather_kernel(token_embedding_table, flat_idx)
    return out.reshape(B, L, D)


# speedup vs baseline: 1.0365x; 1.0215x over previous
import functools

import jax
import jax.numpy as jnp
from jax import lax
from jax.experimental import pallas as pl
from jax.experimental.pallas import tpu as pltpu
from jax.experimental.pallas import tpu_sc as plsc

NUM_WORKERS = 32
CHUNK = 50


def kernel(idx, targets, token_embedding_table):
    del targets
    B, L = idx.shape
    V, D = token_embedding_table.shape
    N = B * L
    per_w = N // NUM_WORKERS
    n_ch = per_w // CHUNK
    assert per_w * NUM_WORKERS == N and n_ch * CHUNK == per_w and n_ch % 2 == 0

    flat_idx = idx.astype(jnp.int32).reshape(NUM_WORKERS, n_ch, CHUNK)

    mesh = plsc.VectorSubcoreMesh(core_axis_name="c", subcore_axis_name="s")

    @functools.partial(
        pl.kernel,
        out_type=jax.ShapeDtypeStruct((N, D), jnp.float32),
        mesh=mesh,
        compiler_params=pltpu.CompilerParams(use_tc_tiling_on_sc=False),
        scratch_types=[
            pltpu.VMEM((n_ch, CHUNK), jnp.int32),
            pltpu.VMEM((CHUNK, D), jnp.float32),
            pltpu.VMEM((CHUNK, D), jnp.float32),
            pltpu.SemaphoreType.DMA,
            pltpu.SemaphoreType.DMA,
            pltpu.SemaphoreType.DMA,
            pltpu.SemaphoreType.DMA,
        ],
    )
    def gather_kernel(
        table_hbm, idx_hbm, out_hbm, idx_v, buf0, buf1, g0, g1, w0, w1
    ):
        wid = lax.axis_index("s") * 2 + lax.axis_index("c")
        base = wid * per_w
        pltpu.sync_copy(idx_hbm.at[wid], idx_v)

        def gather(j, buf, sem):
            return pltpu.async_copy(table_hbm.at[idx_v.at[j]], buf, sem)

        def write(j, buf, sem):
            return pltpu.async_copy(
                buf, out_hbm.at[pl.ds(base + j * CHUNK, CHUNK)], sem
            )

        def wait_write(j, buf, sem):
            pltpu.make_async_copy(
                buf, out_hbm.at[pl.ds(base + j * CHUNK, CHUNK)], sem
            ).wait()

        def wait_gather(j, buf, sem):
            pltpu.make_async_copy(table_hbm.at[idx_v.at[j]], buf, sem).wait()

        gather(0, buf0, g0)

        @pl.loop(0, n_ch, step=2)
        def _pair(j):
            @pl.when(j > 0)
            def _():
                wait_write(j - 1, buf1, w1)

            gather(j + 1, buf1, g1)
            wait_gather(j, buf0, g0)
            write(j, buf0, w0)

            @pl.when(j + 2 < n_ch)
            def _():
                wait_write(j, buf0, w0)
                gather(j + 2, buf0, g0)

            wait_gather(j + 1, buf1, g1)
            write(j + 1, buf1, w1)

        wait_write(n_ch - 2, buf0, w0)
        wait_write(n_ch - 1, buf1, w1)

    out = gather_kernel(token_embedding_table, flat_idx)
    return out.reshape(B, L, D)
